# baseline (speedup 1.0000x reference)
import functools

import jax
import jax.numpy as jnp
from jax import lax
from jax.experimental import pallas as pl
from jax.experimental.pallas import tpu as pltpu
from jax.experimental.pallas import tpu_sc as plsc

_F = 26
_V = 100000
_D = 32
_B = 16384

_NW = 32
_WIN = _B // _NW
_NQ = 4

_mesh = plsc.VectorSubcoreMesh(core_axis_name="c", subcore_axis_name="s")


@functools.partial(
    pl.kernel,
    mesh=_mesh,
    compiler_params=pltpu.CompilerParams(needs_layout_passes=False),
    out_type=jax.ShapeDtypeStruct((_F, _D, _B), jnp.float32),
    scratch_types=[
        pltpu.VMEM((2, _NQ, 128), jnp.int32),
        pltpu.VMEM((2, _WIN), jnp.int32),
        pltpu.VMEM((4, 128, 128), jnp.float32),
        pltpu.VMEM((_NQ, _NQ, 8, 128), jnp.float32),
        pltpu.SemaphoreType.DMA,
        pltpu.SemaphoreType.DMA,
        pltpu.SemaphoreType.DMA,
        pltpu.SemaphoreType.DMA,
        pltpu.SemaphoreType.DMA,
        pltpu.SemaphoreType.DMA,
    ],
)
def _embed_kernel(q_hbm, s_hbm, table_hbm, out_hbm,
                  qbuf, sbuf, gbuf, tbuf, gsem0, gsem1, gsem2, gsem3,
                  osem, xsem):
    wid = lax.axis_index("s") * 2 + lax.axis_index("c")
    b0 = wid * _WIN
    gsem = (gsem0, gsem1, gsem2, gsem3)
    iota16 = lax.iota(jnp.int32, 16)

    def stage_idx(f, issue):
        p = f & 1
        mk = pltpu.async_copy if issue else pltpu.make_async_copy
        descs = [mk(q_hbm.at[f, pl.ds(b0 + j * 128, 128)], qbuf.at[p, j], xsem)
                 for j in range(_NQ)]
        descs.append(mk(s_hbm.at[f, pl.ds(b0, _WIN)], sbuf.at[p], xsem))
        return descs

    def gather(f, qt, issue):
        p = f & 1
        mk = pltpu.async_copy if issue else pltpu.make_async_copy
        return mk(table_hbm.at[qbuf.at[p, qt]], gbuf.at[qt], gsem[qt])

    def out_writes(f, issue):
        mk = pltpu.async_copy if issue else pltpu.make_async_copy
        return [mk(tbuf.at[dg, tb],
                   out_hbm.at[f, pl.ds(dg * 8, 8), pl.ds(b0 + tb * 128, 128)],
                   osem)
                for dg in range(4) for tb in range(4)]

    def extract_quarter(f, qt):
        p = f & 1
        eb = gbuf.at[qt]

        def pack_body(m, _):
            row = m * 16 + iota16
            sub = sbuf[p, pl.ds(qt * 128 + m * 16, 16)]
            for dg in range(4):
                for dd in range(8):
                    col = sub + (dg * 8 + dd)
                    val = plsc.load_gather(eb, [row, col])
                    tbuf[dg, qt, dd, pl.ds(m * 16, 16)] = val
            return 0

        lax.fori_loop(0, 8, pack_body, 0)

    for d in stage_idx(0, True):
        d.wait()
    for qt in range(_NQ):
        gather(0, qt, True)
    stage_idx(1, True)

    def field_body(f, _):
        @pl.when(f + 1 < _F)
        def _():
            for d in stage_idx(f + 1, False):
                d.wait()

        for qt in range(_NQ):
            gather(f, qt, False).wait()
            if qt == 0:
                @pl.when(f >= 1)
                def _():
                    for d in out_writes(f - 1, False):
                        d.wait()
            extract_quarter(f, qt)
            @pl.when(f + 1 < _F)
            def _(qt=qt):
                gather(f + 1, qt, True)

        @pl.when(f + 2 < _F)
        def _():
            stage_idx(f + 2, True)
        out_writes(f, True)
        return 0

    lax.fori_loop(0, _F, field_body, 0)
    for d in out_writes(_F - 1, False):
        d.wait()


def kernel(X, tables):
    xt = jnp.transpose(X).astype(jnp.int32)
    offs = (jnp.arange(_F, dtype=jnp.int32) * (_V // 4))[:, None]
    q = (xt >> 2) + offs
    s = (xt & 3) * _D
    table4 = tables.reshape(_F * _V // 4, 128)
    out = _embed_kernel(q, s, table4)
    return jnp.transpose(out, (2, 0, 1))


# speedup vs baseline: 1.0474x; 1.0474x over previous
import functools

import jax
import jax.numpy as jnp
from jax import lax
from jax.experimental import pallas as pl
from jax.experimental.pallas import tpu as pltpu
from jax.experimental.pallas import tpu_sc as plsc

_F = 26
_V = 100000
_D = 32
_B = 16384

_NW = 32
_ROWS = _B * _F
_RPW = _ROWS // _NW
_IDXW = 128
_CHUNK = 1024
_NCHUNK = _RPW // _CHUNK
_SPC = _CHUNK // _IDXW

_mesh = plsc.VectorSubcoreMesh(core_axis_name="c", subcore_axis_name="s")


@functools.partial(
    pl.kernel,
    mesh=_mesh,
    compiler_params=pltpu.CompilerParams(use_tc_tiling_on_sc=False),
    out_type=jax.ShapeDtypeStruct((_ROWS, _D), jnp.float32),
    scratch_types=[
        pltpu.VMEM((_RPW // _IDXW, _IDXW), jnp.int32),
        pltpu.VMEM((_CHUNK, _D), jnp.float32),
        pltpu.VMEM((_CHUNK, _D), jnp.float32),
        pltpu.SemaphoreType.DMA,
        pltpu.SemaphoreType.DMA,
        pltpu.SemaphoreType.DMA,
        pltpu.SemaphoreType.DMA,
    ],
)
def _gather_kernel(table_hbm, idx_hbm, out_hbm,
                   idx_v, rows0, rows1, gsem0, gsem1, osem0, osem1):
    wid = lax.axis_index("s") * 2 + lax.axis_index("c")
    base = wid * _RPW

    rows = (rows0, rows1)
    gsem = (gsem0, gsem1)
    osem = (osem0, osem1)

    pltpu.sync_copy(idx_hbm.at[pl.ds(wid * (_RPW // _IDXW), _RPW // _IDXW)],
                    idx_v)

    def start_gather(c):
        b = c & 1
        descs = []
        for r in range(_SPC):
            row = c * _SPC + r
            descs.append(pltpu.async_copy(
                table_hbm.at[idx_v.at[row]],
                rows[b].at[pl.ds(r * _IDXW, _IDXW)],
                gsem[b],
            ))
        return descs

    gwait = [None] * _NCHUNK
    owait = [None] * _NCHUNK

    gwait[0] = start_gather(0)
    for c in range(_NCHUNK):
        b = c & 1
        if c + 1 < _NCHUNK:
            if c >= 1:
                owait[c - 1].wait()
            gwait[c + 1] = start_gather(c + 1)
        for d in gwait[c]:
            d.wait()
        owait[c] = pltpu.async_copy(
            rows[b],
            out_hbm.at[pl.ds(base + c * _CHUNK, _CHUNK)],
            osem[b],
        )
    owait[_NCHUNK - 2].wait()
    owait[_NCHUNK - 1].wait()


def kernel(X, tables):
    offs = (jnp.arange(_F, dtype=jnp.int32) * _V)[None, :]
    flat_idx = (X.astype(jnp.int32) + offs).reshape(_ROWS // _IDXW, _IDXW)
    table2d = tables.reshape(_F * _V, _D)
    out = _gather_kernel(table2d, flat_idx)
    return out.reshape(_B, _F, _D)
